# Initial kernel scaffold; baseline (speedup 1.0000x reference)
import jax
import jax.numpy as jnp
from jax.experimental import pallas as pl


def kernel(x, edge_index, W1, b1, W2, b2):
    raise NotImplementedError("write your pallas kernel here")


# speedup vs baseline: 17.9417x; 17.9417x over previous
import functools

import jax
import jax.numpy as jnp
from jax import lax
from jax.experimental import pallas as pl
from jax.experimental.pallas import tpu as pltpu
from jax.experimental.pallas import tpu_sc as plsc

N_NODES = 10000
N_EDGES = 320000
D_IN = 128
D_HID = 64
D2 = 16

NC = 2
NS = 16
NW = NC * NS
CH = 128
NP = 10240
EP = 327680
NCH = EP // (NW * CH)
RPT = NP // NS

_mesh = plsc.VectorSubcoreMesh(core_axis_name="c", subcore_axis_name="s")
_sc_params = pltpu.CompilerParams(use_tc_tiling_on_sc=False)


def _worker(c, s):
    return s * NC + c


@functools.partial(
    pl.kernel,
    out_type=jax.ShapeDtypeStruct((NC * NP, D2), jnp.float32),
    mesh=_mesh,
    scratch_types=[
        pltpu.VMEM((NCH, CH), jnp.int32),
        pltpu.VMEM((CH, D2), jnp.float32),
        pltpu.VMEM_SHARED((NP, D2), jnp.float32),
        pltpu.SemaphoreType.DMA,
    ],
    compiler_params=_sc_params,
)
def _deg_kernel(dst_hbm, ones_hbm, zeros_hbm, out_hbm, dst_v, msg_v, acc_sh, sem):
    c = lax.axis_index("c")
    s = lax.axis_index("s")
    w = _worker(c, s)
    r0 = s * RPT
    pltpu.sync_copy(dst_hbm.at[pl.ds(w * NCH, NCH)], dst_v)
    pltpu.sync_copy(ones_hbm.at[pl.ds(0, CH)], msg_v)

    @pl.when(c == 0)
    def _():
        pltpu.sync_copy(ones_hbm.at[pl.ds(r0, RPT)], acc_sh.at[pl.ds(r0, RPT)])

    @pl.when(c != 0)
    def _():
        pltpu.sync_copy(zeros_hbm.at[pl.ds(r0, RPT)], acc_sh.at[pl.ds(r0, RPT)])

    plsc.subcore_barrier()

    def body(j, carry):
        pltpu.sync_copy(msg_v, acc_sh.at[dst_v.at[j]], add=True)
        return carry

    lax.fori_loop(0, NCH, body, 0)
    plsc.subcore_barrier()
    pltpu.sync_copy(acc_sh.at[pl.ds(r0, RPT)], out_hbm.at[pl.ds(c * NP + r0, RPT)])


def _make_seg_kernel(d):
    @functools.partial(
        pl.kernel,
        out_type=jax.ShapeDtypeStruct((NC * NP, d), jnp.float32),
        mesh=_mesh,
        scratch_types=[
            pltpu.VMEM((NCH, CH), jnp.int32),
            pltpu.VMEM((NCH, CH), jnp.int32),
            pltpu.VMEM((CH, d), jnp.float32),
            pltpu.VMEM_SHARED((NP, d), jnp.float32),
            pltpu.SemaphoreType.DMA,
        ],
        compiler_params=_sc_params,
    )
    def _seg(src_hbm, dst_hbm, tbl_hbm, zeros_hbm, out_hbm,
             src_v, dst_v, msg_v, acc_sh, gsem):
        c = lax.axis_index("c")
        s = lax.axis_index("s")
        w = _worker(c, s)
        r0 = s * RPT
        pltpu.sync_copy(src_hbm.at[pl.ds(w * NCH, NCH)], src_v)
        pltpu.sync_copy(dst_hbm.at[pl.ds(w * NCH, NCH)], dst_v)

        @pl.when(c == 0)
        def _():
            pltpu.sync_copy(tbl_hbm.at[pl.ds(r0, RPT)], acc_sh.at[pl.ds(r0, RPT)])

        @pl.when(c != 0)
        def _():
            pltpu.sync_copy(zeros_hbm.at[pl.ds(r0, RPT)], acc_sh.at[pl.ds(r0, RPT)])

        plsc.subcore_barrier()

        def body(j, carry):
            pltpu.async_copy(tbl_hbm.at[src_v.at[j]], msg_v, gsem).wait()
            pltpu.sync_copy(msg_v, acc_sh.at[dst_v.at[j]], add=True)
            return carry

        lax.fori_loop(0, NCH, body, 0)
        plsc.subcore_barrier()
        pltpu.sync_copy(acc_sh.at[pl.ds(r0, RPT)], out_hbm.at[pl.ds(c * NP + r0, RPT)])

    return _seg


_seg64 = _make_seg_kernel(D_HID)
_seg16 = _make_seg_kernel(D2)


def _tc1_body(deg_ref, x_ref, w1_ref, dis_ref, h1s_ref):
    deg = deg_ref[:NP, 0:1] + deg_ref[NP:, 0:1]
    dis = lax.rsqrt(deg)
    h1 = jnp.dot(x_ref[...], w1_ref[...], preferred_element_type=jnp.float32)
    dis_ref[...] = jnp.broadcast_to(dis, (NP, D2))
    h1s_ref[...] = dis * h1


def _tc_mid_body(s1_ref, dis_ref, b1_ref, w2_ref, g_ref):
    dis = dis_ref[:, 0:1]
    a1 = s1_ref[:NP, :] + s1_ref[NP:, :]
    h = jnp.maximum(dis * a1 + b1_ref[...], 0.0)
    g_ref[...] = dis * jnp.dot(h, w2_ref[...], preferred_element_type=jnp.float32)


def _tc_final_body(s2_ref, dis_ref, b2_ref, out_ref):
    dis = dis_ref[:, 0:1]
    a2 = s2_ref[:NP, :] + s2_ref[NP:, :]
    out_ref[...] = dis * a2 + b2_ref[...]


_tc1 = pl.pallas_call(
    _tc1_body,
    out_shape=(
        jax.ShapeDtypeStruct((NP, D2), jnp.float32),
        jax.ShapeDtypeStruct((NP, D_HID), jnp.float32),
    ),
)

_tc_mid = pl.pallas_call(
    _tc_mid_body,
    out_shape=jax.ShapeDtypeStruct((NP, D2), jnp.float32),
)

_tc_final = pl.pallas_call(
    _tc_final_body,
    out_shape=jax.ShapeDtypeStruct((NP, D2), jnp.float32),
)


def kernel(x, edge_index, W1, b1, W2, b2):
    ei = edge_index.astype(jnp.int32)
    pad = jnp.full((EP - N_EDGES,), N_NODES, jnp.int32)
    src2d = jnp.concatenate([ei[0], pad]).reshape(EP // CH, CH)
    dst2d = jnp.concatenate([ei[1], pad]).reshape(EP // CH, CH)

    xp = jnp.zeros((NP, D_IN), jnp.float32).at[:N_NODES].set(x)
    w2p = jnp.zeros((D_HID, D2), jnp.float32).at[:, : W2.shape[1]].set(W2)
    b1r = b1.reshape(1, D_HID)
    b2r = jnp.zeros((1, D2), jnp.float32).at[0, : b2.shape[0]].set(b2)

    ones16 = jnp.ones((NP, D2), jnp.float32)
    zeros16 = jnp.zeros((NP, D2), jnp.float32)
    zeros64 = jnp.zeros((NP, D_HID), jnp.float32)

    degp = _deg_kernel(dst2d, ones16, zeros16)
    dis16, h1s = _tc1(degp, xp, W1)
    s1p = _seg64(src2d, dst2d, h1s, zeros64)
    g = _tc_mid(s1p, dis16, b1r, w2p)
    s2p = _seg16(src2d, dst2d, g, zeros16)
    outp = _tc_final(s2p, dis16, b2r)
    return outp[:N_NODES, :2]


# speedup vs baseline: 43.0111x; 2.3973x over previous
import functools

import jax
import jax.numpy as jnp
from jax import lax
from jax.experimental import pallas as pl
from jax.experimental.pallas import tpu as pltpu
from jax.experimental.pallas import tpu_sc as plsc

N_NODES = 10000
N_EDGES = 320000
D_IN = 128
D_HID = 64
D2 = 16

NC = 2
NS = 16
NW = NC * NS
CH = 128
NP = 10240
EP = 327680
NCH = EP // (NW * CH)
RPT = NP // NS

_mesh = plsc.VectorSubcoreMesh(core_axis_name="c", subcore_axis_name="s")
_sc_params = pltpu.CompilerParams(use_tc_tiling_on_sc=False)


def _worker(c, s):
    return s * NC + c


@functools.partial(
    pl.kernel,
    out_type=jax.ShapeDtypeStruct((NC * NP, D2), jnp.float32),
    mesh=_mesh,
    scratch_types=[
        pltpu.VMEM((NCH, CH), jnp.int32),
        pltpu.VMEM((CH, D2), jnp.float32),
        pltpu.VMEM_SHARED((NP, D2), jnp.float32),
        pltpu.SemaphoreType.DMA,
    ],
    compiler_params=_sc_params,
)
def _deg_kernel(dst_hbm, ones_hbm, zeros_hbm, out_hbm, dst_v, msg_v, acc_sh, sem):
    c = lax.axis_index("c")
    s = lax.axis_index("s")
    w = _worker(c, s)
    r0 = s * RPT
    pltpu.sync_copy(dst_hbm.at[pl.ds(w * NCH, NCH)], dst_v)
    pltpu.sync_copy(ones_hbm.at[pl.ds(0, CH)], msg_v)

    @pl.when(c == 0)
    def _():
        pltpu.sync_copy(ones_hbm.at[pl.ds(r0, RPT)], acc_sh.at[pl.ds(r0, RPT)])

    @pl.when(c != 0)
    def _():
        pltpu.sync_copy(zeros_hbm.at[pl.ds(r0, RPT)], acc_sh.at[pl.ds(r0, RPT)])

    plsc.subcore_barrier()

    def body(j, carry):
        pltpu.sync_copy(msg_v, acc_sh.at[dst_v.at[j]], add=True)
        return carry

    lax.fori_loop(0, NCH, body, 0)
    plsc.subcore_barrier()
    pltpu.sync_copy(acc_sh.at[pl.ds(r0, RPT)], out_hbm.at[pl.ds(c * NP + r0, RPT)])


def _make_seg_kernel(d):
    @functools.partial(
        pl.kernel,
        out_type=jax.ShapeDtypeStruct((NC * NP, d), jnp.float32),
        mesh=_mesh,
        scratch_types=[
            pltpu.VMEM((NCH, CH), jnp.int32),
            pltpu.VMEM((NCH, CH), jnp.int32),
            pltpu.VMEM((2, CH, d), jnp.float32),
            pltpu.VMEM_SHARED((NP, d), jnp.float32),
            pltpu.SemaphoreType.DMA((2,)),
        ],
        compiler_params=_sc_params,
    )
    def _seg(src_hbm, dst_hbm, tbl_hbm, zeros_hbm, out_hbm,
             src_v, dst_v, msg_v, acc_sh, gsem):
        c = lax.axis_index("c")
        s = lax.axis_index("s")
        w = _worker(c, s)
        r0 = s * RPT
        pltpu.sync_copy(src_hbm.at[pl.ds(w * NCH, NCH)], src_v)
        pltpu.sync_copy(dst_hbm.at[pl.ds(w * NCH, NCH)], dst_v)

        @pl.when(c == 0)
        def _():
            pltpu.sync_copy(tbl_hbm.at[pl.ds(r0, RPT)], acc_sh.at[pl.ds(r0, RPT)])

        @pl.when(c != 0)
        def _():
            pltpu.sync_copy(zeros_hbm.at[pl.ds(r0, RPT)], acc_sh.at[pl.ds(r0, RPT)])

        plsc.subcore_barrier()

        pltpu.async_copy(tbl_hbm.at[src_v.at[0]], msg_v.at[0], gsem.at[0])

        def body(j, carry):
            p = lax.rem(j, 2)
            q = 1 - p

            @pl.when(j + 1 < NCH)
            def _():
                pltpu.async_copy(tbl_hbm.at[src_v.at[j + 1]], msg_v.at[q],
                                 gsem.at[q])

            pltpu.make_async_copy(tbl_hbm.at[src_v.at[j]], msg_v.at[p],
                                  gsem.at[p]).wait()
            pltpu.sync_copy(msg_v.at[p], acc_sh.at[dst_v.at[j]], add=True)
            return carry

        lax.fori_loop(0, NCH, body, 0)
        plsc.subcore_barrier()
        pltpu.sync_copy(acc_sh.at[pl.ds(r0, RPT)], out_hbm.at[pl.ds(c * NP + r0, RPT)])

    return _seg


_seg64 = _make_seg_kernel(D_HID)
_seg16 = _make_seg_kernel(D2)


def _tc1_body(deg_ref, x_ref, w1_ref, dis_ref, h1s_ref):
    deg = deg_ref[:NP, 0:1] + deg_ref[NP:, 0:1]
    dis = lax.rsqrt(deg)
    h1 = jnp.dot(x_ref[...], w1_ref[...], preferred_element_type=jnp.float32)
    dis_ref[...] = jnp.broadcast_to(dis, (NP, D2))
    h1s_ref[...] = dis * h1


def _tc_mid_body(s1_ref, dis_ref, b1_ref, w2_ref, g_ref):
    dis = dis_ref[:, 0:1]
    a1 = s1_ref[:NP, :] + s1_ref[NP:, :]
    h = jnp.maximum(dis * a1 + b1_ref[...], 0.0)
    g_ref[...] = dis * jnp.dot(h, w2_ref[...], preferred_element_type=jnp.float32)


def _tc_final_body(s2_ref, dis_ref, b2_ref, out_ref):
    dis = dis_ref[:, 0:1]
    a2 = s2_ref[:NP, :] + s2_ref[NP:, :]
    out_ref[...] = dis * a2 + b2_ref[...]


_tc1 = pl.pallas_call(
    _tc1_body,
    out_shape=(
        jax.ShapeDtypeStruct((NP, D2), jnp.float32),
        jax.ShapeDtypeStruct((NP, D_HID), jnp.float32),
    ),
)

_tc_mid = pl.pallas_call(
    _tc_mid_body,
    out_shape=jax.ShapeDtypeStruct((NP, D2), jnp.float32),
)

_tc_final = pl.pallas_call(
    _tc_final_body,
    out_shape=jax.ShapeDtypeStruct((NP, D2), jnp.float32),
)


def kernel(x, edge_index, W1, b1, W2, b2):
    ei = edge_index.astype(jnp.int32)
    pad = N_NODES + jnp.arange(EP - N_EDGES, dtype=jnp.int32) % (NP - N_NODES)
    src2d = jnp.concatenate([ei[0], pad]).reshape(EP // CH, CH)
    dst2d = jnp.concatenate([ei[1], pad]).reshape(EP // CH, CH)

    xp = jnp.zeros((NP, D_IN), jnp.float32).at[:N_NODES].set(x)
    w2p = jnp.zeros((D_HID, D2), jnp.float32).at[:, : W2.shape[1]].set(W2)
    b1r = b1.reshape(1, D_HID)
    b2r = jnp.zeros((1, D2), jnp.float32).at[0, : b2.shape[0]].set(b2)

    ones16 = jnp.ones((NP, D2), jnp.float32)
    zeros16 = jnp.zeros((NP, D2), jnp.float32)
    zeros64 = jnp.zeros((NP, D_HID), jnp.float32)

    degp = _deg_kernel(dst2d, ones16, zeros16)
    dis16, h1s = _tc1(degp, xp, W1)
    s1p = _seg64(src2d, dst2d, h1s, zeros64)
    g = _tc_mid(s1p, dis16, b1r, w2p)
    s2p = _seg16(src2d, dst2d, g, zeros16)
    outp = _tc_final(s2p, dis16, b2r)
    return outp[:N_NODES, :2]
